# baseline (speedup 1.0000x reference)
import functools

import jax
import jax.numpy as jnp
from jax import lax
from jax.experimental import pallas as pl
from jax.experimental.pallas import tpu as pltpu
from jax.experimental.pallas import tpu_sc as plsc

B, S, H = 1024, 200, 64
BS = B * S
NC, NS = 2, 16
NW = NC * NS
ROWS_PER_W = BS // NW
CHUNK = 128
NCHUNK = ROWS_PER_W // CHUNK


def _body(ids_hbm, scal_hbm, node_hbm, label_hbm, out_hbm, *scratch):
    ids_v = scratch[0:3]
    scal_v = scratch[3:6]
    gb_v = [scratch[6:9], scratch[9:12]]
    st_v = scratch[12:14]
    sem_in = scratch[14:17]
    sem_g = scratch[17:19]
    sem_out = scratch[19:21]

    wid = lax.axis_index("s") * NC + lax.axis_index("c")

    def in_descs(c, b3):
        cg = wid * NCHUNK + c
        return [
            pltpu.make_async_copy(ids_hbm.at[:, pl.ds(cg, 1), :], ids_v[b3],
                                  sem_in[b3]),
            pltpu.make_async_copy(scal_hbm.at[:, pl.ds(cg * CHUNK, CHUNK)],
                                  scal_v[b3], sem_in[b3]),
        ]

    def g_descs(c, b3, b2):
        return [
            pltpu.make_async_copy(node_hbm.at[ids_v[b3].at[0, 0]],
                                  gb_v[b2][0], sem_g[b2]),
            pltpu.make_async_copy(node_hbm.at[ids_v[b3].at[1, 0]],
                                  gb_v[b2][1], sem_g[b2]),
            pltpu.make_async_copy(label_hbm.at[ids_v[b3].at[2, 0]],
                                  gb_v[b2][2], sem_g[b2]),
        ]

    def out_descs(c, b2):
        rows = pl.ds((wid * NCHUNK + c) * CHUNK, CHUNK)
        return [
            pltpu.make_async_copy(st_v[b2], out_hbm.at[rows, :], sem_out[b2]),
        ]

    def fire(descs):
        for d in descs:
            d.start()

    def drain(descs):
        for d in descs:
            d.wait()

    def compute(b3, b2):
        srows, drows, lrows = gb_v[b2]
        stage = st_v[b2]
        scal = scal_v[b3]
        i0 = jnp.full((16,), 0, jnp.int32)
        i1 = jnp.full((16,), 1, jnp.int32)
        i2 = jnp.full((16,), 2, jnp.int32)
        i3 = jnp.full((16,), 3, jnp.int32)

        def row_body(r, _):
            ridx = jnp.full((16,), r, jnp.int32)
            et = plsc.load_gather(scal, [i0, ridx])
            sm = plsc.load_gather(scal, [i1, ridx])
            dm = plsc.load_gather(scal, [i2, ridx])
            lm = plsc.load_gather(scal, [i3, ridx])
            for q in range(H // 16):
                sl = pl.ds(q * 16, 16)
                stage[r, pl.ds(q * 16, 16)] = et
                stage[r, pl.ds(H + q * 16, 16)] = srows[r, sl] * sm
                stage[r, pl.ds(2 * H + q * 16, 16)] = drows[r, sl] * dm
                stage[r, pl.ds(3 * H + q * 16, 16)] = lrows[r, sl] * lm
            return _

        lax.fori_loop(0, CHUNK, row_body, None, unroll=8)

    def iter_ops(c, b3, b2, *, out_wait=True, fire_next_g=True,
                 fire_next_in=True):
        drain(g_descs(c, b3, b2))
        if fire_next_g:
            drain(in_descs(c + 1, (b3 + 1) % 3))
            fire(g_descs(c + 1, (b3 + 1) % 3, (b2 + 1) % 2))
        if out_wait:
            drain(out_descs(c - 2, b2))
        compute(b3, b2)
        fire(out_descs(c, b2))
        if fire_next_in:
            fire(in_descs(c + 2, (b3 + 2) % 3))

    fire(in_descs(0, 0))
    fire(in_descs(1, 1))
    drain(in_descs(0, 0))
    fire(g_descs(0, 0, 0))

    for c in range(6):
        iter_ops(c, c % 3, c % 2, out_wait=(c >= 2))

    n_steady = NCHUNK - 8
    def outer(cc, _):
        for j in range(6):
            iter_ops(6 + cc * 6 + j, j % 3, j % 2)
        return _

    lax.fori_loop(0, n_steady // 6, outer, None)

    iter_ops(NCHUNK - 2, (NCHUNK - 2) % 3, (NCHUNK - 2) % 2,
             fire_next_in=False)
    c = NCHUNK - 1
    drain(g_descs(c, c % 3, c % 2))
    drain(out_descs(c - 2, c % 2))
    compute(c % 3, c % 2)
    fire(out_descs(c, c % 2))
    drain(out_descs(NCHUNK - 2, (NCHUNK - 2) % 2))
    drain(out_descs(NCHUNK - 1, c % 2))


@jax.jit
def _encode(ids, scal, node_emb, label_emb):
    mesh = plsc.VectorSubcoreMesh(core_axis_name="c", subcore_axis_name="s")
    scratch = (
        [pltpu.VMEM((3, 1, CHUNK), jnp.int32) for _ in range(3)]
        + [pltpu.VMEM((4, CHUNK), jnp.float32) for _ in range(3)]
        + [pltpu.VMEM((CHUNK, H), jnp.float32) for _ in range(6)]
        + [pltpu.VMEM((CHUNK, 4 * H), jnp.float32) for _ in range(2)]
        + [pltpu.SemaphoreType.DMA for _ in range(7)]
    )
    f = functools.partial(
        pl.kernel,
        out_type=jax.ShapeDtypeStruct((BS, 4 * H), jnp.float32),
        mesh=mesh,
        compiler_params=pltpu.CompilerParams(use_tc_tiling_on_sc=False,
                                             needs_layout_passes=False),
        scratch_types=scratch,
    )(_body)
    return f(ids, scal, node_emb, label_emb)


def kernel(event_type_id, src_id, src_mask, dst_id, dst_mask, label_id,
           label_mask, node_embeddings, label_embeddings):
    ids = jnp.stack([src_id.astype(jnp.int32).reshape(BS),
                     dst_id.astype(jnp.int32).reshape(BS),
                     label_id.astype(jnp.int32).reshape(BS)]
                    ).reshape(3, BS // CHUNK, CHUNK)
    scal = jnp.stack([event_type_id.reshape(BS),
                      src_mask.reshape(BS),
                      dst_mask.reshape(BS),
                      label_mask.reshape(BS)])
    out = _encode(ids, scal, node_embeddings, label_embeddings)
    return out.reshape(B, S, 4 * H)


# speedup vs baseline: 1.5798x; 1.5798x over previous
import functools

import jax
import jax.numpy as jnp
from jax import lax
from jax.experimental import pallas as pl
from jax.experimental.pallas import tpu as pltpu
from jax.experimental.pallas import tpu_sc as plsc

B, S, H = 1024, 200, 64
BS = B * S
NC, NS = 2, 16
NW = NC * NS
ROWS_PER_W = BS // NW
CHUNK = 128
NCHUNK = ROWS_PER_W // CHUNK
NBUF = 3


def _body(ids_hbm, scal_hbm, node_hbm, label_hbm, out_hbm, *scratch):
    ids_v = scratch[0:3]
    scal_v = scratch[3:6]
    rows_v = [scratch[6 + 4 * b:6 + 4 * b + 4] for b in range(3)]
    sem_in = scratch[18:21]
    sem_g = scratch[21:24]
    sem_out = scratch[24:27]

    wid = lax.axis_index("s") * NC + lax.axis_index("c")

    def in_descs(c, b):
        cg = wid * NCHUNK + c
        return [
            pltpu.make_async_copy(ids_hbm.at[:, pl.ds(cg, 1), :], ids_v[b],
                                  sem_in[b]),
            pltpu.make_async_copy(scal_hbm.at[:, pl.ds(cg * CHUNK, CHUNK)],
                                  scal_v[b], sem_in[b]),
        ]

    def g_descs(c, b):
        return [
            pltpu.make_async_copy(node_hbm.at[ids_v[b].at[0, 0]],
                                  rows_v[b][1], sem_g[b]),
            pltpu.make_async_copy(node_hbm.at[ids_v[b].at[1, 0]],
                                  rows_v[b][2], sem_g[b]),
            pltpu.make_async_copy(label_hbm.at[ids_v[b].at[2, 0]],
                                  rows_v[b][3], sem_g[b]),
        ]

    def out_descs(c, b):
        rows = pl.ds((wid * NCHUNK + c) * CHUNK, CHUNK)
        return [
            pltpu.make_async_copy(rows_v[b][q],
                                  out_hbm.at[rows, pl.ds(q * H, H)],
                                  sem_out[b])
            for q in range(4)
        ]

    def fire(descs):
        for d in descs:
            d.start()

    def drain(descs):
        for d in descs:
            d.wait()

    def compute(b):
        etblk, srows, drows, lrows = rows_v[b]
        scal = scal_v[b]
        i0 = jnp.full((16,), 0, jnp.int32)
        i1 = jnp.full((16,), 1, jnp.int32)
        i2 = jnp.full((16,), 2, jnp.int32)
        i3 = jnp.full((16,), 3, jnp.int32)

        nq = H // 16

        def row_body(r, _):
            ridx = jnp.full((16,), r, jnp.int32)
            et = plsc.load_gather(scal, [i0, ridx])
            sm = plsc.load_gather(scal, [i1, ridx])
            dm = plsc.load_gather(scal, [i2, ridx])
            lm = plsc.load_gather(scal, [i3, ridx])
            ss = [srows[r, pl.ds(q * 16, 16)] for q in range(nq)]
            dd = [drows[r, pl.ds(q * 16, 16)] for q in range(nq)]
            ll = [lrows[r, pl.ds(q * 16, 16)] for q in range(nq)]
            for q in range(nq):
                etblk[r, pl.ds(q * 16, 16)] = et
            for q in range(nq):
                srows[r, pl.ds(q * 16, 16)] = ss[q] * sm
            for q in range(nq):
                drows[r, pl.ds(q * 16, 16)] = dd[q] * dm
            for q in range(nq):
                lrows[r, pl.ds(q * 16, 16)] = ll[q] * lm
            return _

        lax.fori_loop(0, CHUNK, row_body, None, unroll=4)

    def iter_ops(c, b, *, first_out_wait=True, fire_next_g=True,
                 fire_next_in=True):
        drain(g_descs(c, b))
        if fire_next_g:
            bn = (b + 1) % NBUF
            drain(in_descs(c + 1, bn))
            if first_out_wait:
                drain(out_descs(c - 2, bn))
            fire(g_descs(c + 1, bn))
        compute(b)
        fire(out_descs(c, b))
        if fire_next_in:
            fire(in_descs(c + 2, (b + 2) % NBUF))

    fire(in_descs(0, 0))
    fire(in_descs(1, 1))
    drain(in_descs(0, 0))
    fire(g_descs(0, 0))

    iter_ops(0, 0, first_out_wait=False)
    iter_ops(1, 1, first_out_wait=False)
    iter_ops(2, 2)

    n_steady = NCHUNK - 5
    def outer(cc, _):
        for j in range(NBUF):
            iter_ops(3 + cc * NBUF + j, j)
        return _

    lax.fori_loop(0, n_steady // NBUF, outer, None)

    iter_ops(NCHUNK - 2, (NCHUNK - 2) % NBUF, fire_next_in=False)
    c = NCHUNK - 1
    b = c % NBUF
    drain(g_descs(c, b))
    drain(out_descs(c - 2, (b + 1) % NBUF))
    compute(b)
    fire(out_descs(c, b))
    drain(out_descs(NCHUNK - 2, (NCHUNK - 2) % NBUF))
    drain(out_descs(NCHUNK - 1, b))


@jax.jit
def _encode(ids, scal, node_emb, label_emb):
    mesh = plsc.VectorSubcoreMesh(core_axis_name="c", subcore_axis_name="s")
    scratch = (
        [pltpu.VMEM((3, 1, CHUNK), jnp.int32) for _ in range(NBUF)]
        + [pltpu.VMEM((4, CHUNK), jnp.float32) for _ in range(NBUF)]
        + [pltpu.VMEM((CHUNK, H), jnp.float32) for _ in range(4 * NBUF)]
        + [pltpu.SemaphoreType.DMA for _ in range(3 * NBUF)]
    )
    f = functools.partial(
        pl.kernel,
        out_type=jax.ShapeDtypeStruct((BS, 4 * H), jnp.float32),
        mesh=mesh,
        compiler_params=pltpu.CompilerParams(use_tc_tiling_on_sc=False,
                                             needs_layout_passes=False),
        scratch_types=scratch,
    )(_body)
    return f(ids, scal, node_emb, label_emb)


def kernel(event_type_id, src_id, src_mask, dst_id, dst_mask, label_id,
           label_mask, node_embeddings, label_embeddings):
    ids = jnp.stack([src_id.astype(jnp.int32).reshape(BS),
                     dst_id.astype(jnp.int32).reshape(BS),
                     label_id.astype(jnp.int32).reshape(BS)]
                    ).reshape(3, BS // CHUNK, CHUNK)
    scal = jnp.stack([event_type_id.reshape(BS),
                      src_mask.reshape(BS),
                      dst_mask.reshape(BS),
                      label_mask.reshape(BS)])
    out = _encode(ids, scal, node_embeddings, label_embeddings)
    return out.reshape(B, S, 4 * H)
